# baseline (speedup 1.0000x reference)
import jax
import jax.numpy as jnp
from jax import lax
from jax.experimental import pallas as pl
from jax.experimental.pallas import tpu as pltpu
from jax.experimental.pallas import tpu_sc as plsc

_BETA = 0.2

_N = 100000
_K = 50000
_D = 128
_CNT = 3200
_CH = 128
_NCHUNK = _CNT // _CH
_LANES = 16


def _body(z_hbm, nid_hbm, rec_hbm, out_hbm,
          idx_v, w_v, uabs_v, uw_v, zu, ru, zu2, ru2, csem, gsem, ssem):
    c = lax.axis_index("c")
    s = lax.axis_index("s")
    wid = c * 16 + s
    lo = wid * _CNT
    cnt = jnp.minimum(_CNT, _N - lo)

    last = wid == 31

    idx_cp = pltpu.make_async_copy(nid_hbm, idx_v, csem)
    idx_cp.start()

    lanes = lax.iota(jnp.int32, _LANES)

    def memset(k, carry):
        w_v[pl.ds(k * _LANES, _LANES)] = jnp.full((_LANES,), -1, jnp.int32)
        return carry
    lax.fori_loop(0, _CNT // _LANES, memset, 0, unroll=4)
    idx_cp.wait()

    def scan(v, carry):
        idx = idx_v[pl.ds(v * _LANES, _LANES)]
        rel = idx - lo
        m = (rel >= 0) & (rel < cnt)
        _, lastm = plsc.scan_count(rel, mask=m)
        sm = m & lastm
        relc = jnp.where(sm, rel, 0)
        iv = v * _LANES + lanes
        plsc.store_scatter(w_v, [relc], iv, mask=sm)
        return carry
    lax.fori_loop(0, _K // _LANES, scan, 0, unroll=25)

    def compact(k, total):
        wv = w_v[pl.ds(k * _LANES, _LANES)]
        m = wv >= 0
        mi = jnp.where(m, 1, 0).astype(jnp.int32)
        incl = plsc.cumsum(mi)
        pos = total + incl - mi
        hi = pos >> 7
        lje = pos & (_CH - 1)
        rel = k * _LANES + lanes
        plsc.store_scatter(uabs_v, [hi, lje], rel + lo, mask=m)
        plsc.store_scatter(uw_v, [hi, lje], wv, mask=m)
        return total + incl[_LANES - 1]
    ucount = lax.fori_loop(0, _CNT // _LANES, compact, jnp.int32(0), unroll=4)

    e0a = uabs_v[0, pl.ds(0, _LANES)]
    e0w = uw_v[0, pl.ds(0, _LANES)]
    pad_abs = jnp.full((_LANES,), 0, jnp.int32) + e0a[0]
    pad_w = jnp.full((_LANES,), 0, jnp.int32) + e0w[0]

    def padfill(k, carry):
        base = ucount + k * _LANES
        hi = (base + lanes) >> 7
        lje = (base + lanes) & (_CH - 1)
        m = (base + lanes) < ((ucount + _CH - 1) & ~jnp.int32(_CH - 1))
        plsc.store_scatter(uabs_v, [hi, lje], pad_abs, mask=m)
        plsc.store_scatter(uw_v, [hi, lje], pad_w, mask=m)
        return carry
    lax.fori_loop(0, _CH // _LANES, padfill, 0)

    nbf = cnt >> 7

    def cpin(b, buf):
        return pltpu.make_async_copy(
            z_hbm.at[pl.ds(lo + b * _CH, _CH)], buf, csem)

    cpin(0, zu).start()

    def copyblk(i, carry):
        b = i * 2

        @pl.when(b < nbf)
        def _():
            cpin(b, zu).wait()

            @pl.when(b + 1 < nbf)
            def _():
                cpin(b + 1, ru).start()
            pltpu.sync_copy(zu, out_hbm.at[pl.ds(lo + b * _CH, _CH)])

        @pl.when(b + 1 < nbf)
        def _():
            cpin(b + 1, ru).wait()

            @pl.when(b + 2 < nbf)
            def _():
                cpin(b + 2, zu).start()
            pltpu.sync_copy(ru, out_hbm.at[pl.ds(lo + (b + 1) * _CH, _CH)])
        return carry
    lax.fori_loop(0, (nbf + 1) >> 1, copyblk, 0)

    @pl.when(last)
    def _():
        pltpu.sync_copy(z_hbm.at[pl.ds(lo + 768, 32)], zu.at[pl.ds(0, 32)])
        pltpu.sync_copy(zu.at[pl.ds(0, 32)], out_hbm.at[pl.ds(lo + 768, 32)])

    nch = (ucount + _CH - 1) >> 7

    def start_gathers(ch, zb, rb):
        pltpu.async_copy(z_hbm.at[uabs_v.at[ch]], zb, gsem)
        pltpu.async_copy(rec_hbm.at[uw_v.at[ch]], rb, gsem)

    def wait_gathers(ch, zb, rb):
        pltpu.make_async_copy(z_hbm.at[uabs_v.at[ch]], zb, gsem).wait()
        pltpu.make_async_copy(rec_hbm.at[uw_v.at[ch]], rb, gsem).wait()

    def blend(zb, rb):
        def row(r, rcarry):
            for q in range(_D // _LANES):
                zv = zb[r, pl.ds(q * _LANES, _LANES)]
                rv = rb[r, pl.ds(q * _LANES, _LANES)]
                zb[r, pl.ds(q * _LANES, _LANES)] = (
                    zv + jnp.float32(1.0 - _BETA) * (rv - zv))
            return rcarry
        lax.fori_loop(0, _CH, row, 0)

    def wait_scatter(ch, zb):
        pltpu.make_async_copy(zb, out_hbm.at[uabs_v.at[ch]], ssem).wait()

    def section(ch, zb, rb, zb_nxt, rb_nxt):
        wait_gathers(ch, zb, rb)

        @pl.when(ch >= 1)
        def _():
            wait_scatter(ch - 1, zb_nxt)

        @pl.when(ch + 1 < nch)
        def _():
            start_gathers(ch + 1, zb_nxt, rb_nxt)
        blend(zb, rb)
        pltpu.async_copy(zb, out_hbm.at[uabs_v.at[ch]], ssem)

    @pl.when(nch > 0)
    def _():
        start_gathers(0, zu, ru)

    def chunkpair(i, carry):
        ch = i * 2

        @pl.when(ch < nch)
        def _():
            section(ch, zu, ru, zu2, ru2)

        @pl.when(ch + 1 < nch)
        def _():
            section(ch + 1, zu2, ru2, zu, ru)
        return carry
    lax.fori_loop(0, (nch + 1) >> 1, chunkpair, 0)

    @pl.when((nch & 1) == 1)
    def _():
        wait_scatter(nch - 1, zu)

    @pl.when((nch > 0) & ((nch & 1) == 0))
    def _():
        wait_scatter(nch - 1, zu2)


def kernel(z, nodes_id, records):
    mesh = plsc.VectorSubcoreMesh(
        core_axis_name="c", subcore_axis_name="s", num_cores=2, num_subcores=16
    )
    return pl.kernel(
        _body,
        out_type=jax.ShapeDtypeStruct((_N, _D), jnp.float32),
        mesh=mesh,
        compiler_params=pltpu.CompilerParams(needs_layout_passes=False),
        scratch_types=[
            pltpu.VMEM((_K,), jnp.int32),
            pltpu.VMEM((_CNT,), jnp.int32),
            pltpu.VMEM((_NCHUNK, _CH), jnp.int32),
            pltpu.VMEM((_NCHUNK, _CH), jnp.int32),
            pltpu.VMEM((_CH, _D), jnp.float32),
            pltpu.VMEM((_CH, _D), jnp.float32),
            pltpu.VMEM((_CH, _D), jnp.float32),
            pltpu.VMEM((_CH, _D), jnp.float32),
            pltpu.SemaphoreType.DMA,
            pltpu.SemaphoreType.DMA,
            pltpu.SemaphoreType.DMA,
        ],
    )(z, nodes_id, records)
